# baseline (speedup 1.0000x reference)
import jax
import jax.numpy as jnp
from jax import lax
from jax.experimental import pallas as pl
from jax.experimental.pallas import tpu as pltpu
from jax.experimental.pallas import tpu_sc as plsc

N = 10000
E = 320000
D = 128
R = 6
G = 128
NSL = 8
SL = D // NSL
NK = R * N
NC = 2
NS = 16
LANES = 16
PPC = NSL // NC

_CB = 2000
_C_BLOCKS = (E // (NC * NS)) // _CB
_C_STRIPE = NK // NS


def _counts_body(dst_h, et_h, out_h, dstb, etb, keyb, ones_v, zb, cacc):
    c = lax.axis_index("c")
    s = lax.axis_index("s")

    zeros16 = jnp.zeros((LANES,), jnp.float32)
    ones16 = jnp.ones((LANES,), jnp.float32)

    def _fill(i, _):
        zb[i, :] = zeros16
        return ()
    lax.fori_loop(0, zb.shape[0], _fill, ())

    def _fill1(i, _):
        ones_v[i, :] = ones16
        return ()
    lax.fori_loop(0, _CB, _fill1, ())

    for k in range(_C_STRIPE // zb.shape[0]):
        pltpu.sync_copy(zb, cacc.at[pl.ds(s * _C_STRIPE + k * zb.shape[0],
                                          zb.shape[0])])
    plsc.subcore_barrier()

    def _block(b, _):
        base = pl.multiple_of(c * (E // 2) + s * (E // (NC * NS)) + b * _CB, 8)
        pltpu.sync_copy(dst_h.at[pl.ds(base, _CB)], dstb)
        pltpu.sync_copy(et_h.at[pl.ds(base, _CB)], etb)

        def _vec(j, _):
            d16 = dstb[pl.ds(j * LANES, LANES)]
            e16 = etb[pl.ds(j * LANES, LANES)]
            keyb[pl.ds(j * LANES, LANES)] = e16 * N + d16
            return ()
        lax.fori_loop(0, _CB // LANES, _vec, ())

        pltpu.sync_copy(ones_v, cacc.at[keyb], add=True)
        return ()
    lax.fori_loop(0, _C_BLOCKS, _block, ())

    plsc.subcore_barrier()

    @pl.when((s == 0) & (c == 0))
    def _():
        pltpu.sync_copy(cacc, out_h.at[0])

    @pl.when((s == 0) & (c == 1))
    def _():
        pltpu.sync_copy(cacc, out_h.at[1])


_counts_fn = pl.kernel(
    _counts_body,
    out_type=jax.ShapeDtypeStruct((NC, NK, LANES), jnp.float32),
    mesh=plsc.VectorSubcoreMesh(core_axis_name="c", subcore_axis_name="s"),
    compiler_params=pltpu.CompilerParams(use_tc_tiling_on_sc=False),
    scratch_types=[
        pltpu.VMEM((_CB,), jnp.int32),
        pltpu.VMEM((_CB,), jnp.int32),
        pltpu.VMEM((_CB,), jnp.int32),
        pltpu.VMEM((_CB, LANES), jnp.float32),
        pltpu.VMEM((750, LANES), jnp.float32),
        pltpu.VMEM_SHARED((NK, LANES), jnp.float32),
    ],
)


_EB = 800
_S_BLOCKS = (E // NS) // _EB
_S_PAIRS = (_S_BLOCKS - 1) // 2
_S_STRIPE = NK // NS


def _scatter_body(xs_h, src_h, dst_h, et_h, out_h,
                  srcb, dstb, etb, idxb0, keyb0, idxb1, keyb1,
                  rows0, rows1, zb, acc, sem0, sem1):
    c = lax.axis_index("c")
    s = lax.axis_index("s")

    zeros16 = jnp.zeros((LANES,), jnp.float32)

    def _fill(i, _):
        zb[i, :] = zeros16
        return ()
    lax.fori_loop(0, zb.shape[0], _fill, ())

    for p_local in range(PPC):
        for k in range(_S_STRIPE // zb.shape[0]):
            pltpu.sync_copy(zb, acc.at[pl.ds(s * _S_STRIPE + k * zb.shape[0],
                                             zb.shape[0])])
        plsc.subcore_barrier()

        p = PPC * c + p_local
        col = p * SL
        poff = p * N

        def _load_edges(b):
            base = pl.multiple_of(s * (E // NS) + b * _EB, 8)
            pltpu.sync_copy(src_h.at[pl.ds(base, _EB)], srcb)
            pltpu.sync_copy(dst_h.at[pl.ds(base, _EB)], dstb)
            pltpu.sync_copy(et_h.at[pl.ds(base, _EB)], etb)

        def _compute(idxb, keyb):
            def _vec(j, _):
                s16 = srcb[pl.ds(j * LANES, LANES)]
                d16 = dstb[pl.ds(j * LANES, LANES)]
                e16 = etb[pl.ds(j * LANES, LANES)]
                idxb[pl.ds(j * LANES, LANES)] = s16 + poff
                keyb[pl.ds(j * LANES, LANES)] = e16 * N + d16
                return ()
            lax.fori_loop(0, _EB // LANES, _vec, ())

        _load_edges(0)
        _compute(idxb0, keyb0)

        def _pair(k, _):
            _load_edges(2 * k + 1)
            _compute(idxb1, keyb1)
            pltpu.sync_copy(rows0, acc.at[keyb0], add=True)
            _load_edges(2 * k + 2)
            _compute(idxb0, keyb0)
            pltpu.sync_copy(rows1, acc.at[keyb1], add=True)
            return ()
        lax.fori_loop(0, _S_PAIRS, _pair, ())

        pltpu.sync_copy(rows0, acc.at[keyb0], add=True)

        plsc.subcore_barrier()

        @pl.when(s == 0)
        def _():
            pltpu.sync_copy(acc, out_h.at[:, pl.ds(col, SL)])

        plsc.subcore_barrier()


_scatter_fn = pl.kernel(
    _scatter_body,
    out_type=jax.ShapeDtypeStruct((NK, D), jnp.float32),
    mesh=plsc.VectorSubcoreMesh(core_axis_name="c", subcore_axis_name="s"),
    compiler_params=pltpu.CompilerParams(use_tc_tiling_on_sc=False),
    scratch_types=[
        pltpu.VMEM((_EB,), jnp.int32),
        pltpu.VMEM((_EB,), jnp.int32),
        pltpu.VMEM((_EB,), jnp.int32),
        pltpu.VMEM((_EB,), jnp.int32),
        pltpu.VMEM((_EB,), jnp.int32),
        pltpu.VMEM((_EB,), jnp.int32),
        pltpu.VMEM((_EB,), jnp.int32),
        pltpu.VMEM((_EB, SL), jnp.float32),
        pltpu.VMEM((_EB, SL), jnp.float32),
        pltpu.VMEM((375, SL), jnp.float32),
        pltpu.VMEM_SHARED((NK, SL), jnp.float32),
        pltpu.SemaphoreType.DMA,
        pltpu.SemaphoreType.DMA,
    ],
)


_BN = 1000


def _layer_math(xb, s_ref, c_ref, root_ref, w_ref, b_ref):
    acc = jnp.dot(xb, root_ref[...], preferred_element_type=jnp.float32)
    acc = acc + b_ref[...]
    for r in range(R):
        cnt = c_ref[0, r][:, 0:1] + c_ref[1, r][:, 0:1]
        inv = 1.0 / jnp.maximum(cnt, 1.0)
        acc = acc + jnp.dot(s_ref[r] * inv, w_ref[r],
                            preferred_element_type=jnp.float32)
    return jnp.maximum(acc, 0.0)


def _l1_body(x_ref, s_ref, c_ref, root_ref, w_ref, b_ref, o_ref):
    h = _layer_math(x_ref[...], s_ref, c_ref, root_ref, w_ref, b_ref)
    o_ref[...] = jnp.stack([h[:, q * SL:(q + 1) * SL] for q in range(NSL)],
                           axis=0)


def _l2_body(h_ref, s_ref, c_ref, root_ref, w_ref, b_ref, batch_ref, o_ref,
             sum_s, cnt_s):
    i = pl.program_id(0)
    xb = jnp.concatenate([h_ref[q] for q in range(NSL)], axis=-1)
    h2 = _layer_math(xb, s_ref, c_ref, root_ref, w_ref, b_ref)

    bids = batch_ref[0]
    p_oh = (bids == lax.broadcasted_iota(jnp.int32, (G, _BN), 0))
    p_oh = p_oh.astype(jnp.float32)

    @pl.when(i == 0)
    def _():
        sum_s[...] = jnp.zeros_like(sum_s)
        cnt_s[...] = jnp.zeros_like(cnt_s)

    sum_s[...] += jnp.dot(p_oh, h2, preferred_element_type=jnp.float32)
    cnt_s[...] += jnp.sum(p_oh, axis=1, keepdims=True)

    @pl.when(i == pl.num_programs(0) - 1)
    def _():
        o_ref[...] = sum_s[...] / jnp.maximum(cnt_s[...], 1.0)


def _run_l1(x, s1, cnt, root1, w1, b1):
    return pl.pallas_call(
        _l1_body,
        grid=(N // _BN,),
        in_specs=[
            pl.BlockSpec((_BN, D), lambda i: (i, 0)),
            pl.BlockSpec((R, _BN, D), lambda i: (0, i, 0)),
            pl.BlockSpec((NC, R, _BN, LANES), lambda i: (0, 0, i, 0)),
            pl.BlockSpec((D, D), lambda i: (0, 0)),
            pl.BlockSpec((R, D, D), lambda i: (0, 0, 0)),
            pl.BlockSpec((1, D), lambda i: (0, 0)),
        ],
        out_specs=pl.BlockSpec((NSL, _BN, SL), lambda i: (0, i, 0)),
        out_shape=jax.ShapeDtypeStruct((NSL, N, SL), jnp.float32),
    )(x, s1, cnt, root1, w1, b1)


def _run_l2(hs, s2, cnt, root2, w2, b2, batch3):
    return pl.pallas_call(
        _l2_body,
        grid=(N // _BN,),
        in_specs=[
            pl.BlockSpec((NSL, _BN, SL), lambda i: (0, i, 0)),
            pl.BlockSpec((R, _BN, D), lambda i: (0, i, 0)),
            pl.BlockSpec((NC, R, _BN, LANES), lambda i: (0, 0, i, 0)),
            pl.BlockSpec((D, D), lambda i: (0, 0)),
            pl.BlockSpec((R, D, D), lambda i: (0, 0, 0)),
            pl.BlockSpec((1, D), lambda i: (0, 0)),
            pl.BlockSpec((1, 1, _BN), lambda i: (i, 0, 0)),
        ],
        out_specs=pl.BlockSpec((G, D), lambda i: (0, 0)),
        out_shape=jax.ShapeDtypeStruct((G, D), jnp.float32),
        scratch_shapes=[
            pltpu.VMEM((G, D), jnp.float32),
            pltpu.VMEM((G, 1), jnp.float32),
        ],
    )(hs, s2, cnt, root2, w2, b2, batch3)


def kernel(x, edge_index, edge_type, batch, W1, root1, b1, W2, root2, b2):
    src = edge_index[0].astype(jnp.int32)
    dst = edge_index[1].astype(jnp.int32)
    et = edge_type.astype(jnp.int32)
    batch3 = batch.astype(jnp.int32).reshape(N // _BN, 1, _BN)

    xs = x.reshape(N, NSL, SL).transpose(1, 0, 2).reshape(NSL * N, SL)

    cnt = _counts_fn(dst, et)
    cnt = cnt.reshape(NC, R, N, LANES)

    s1 = _scatter_fn(xs, src, dst, et)
    s1 = s1.reshape(R, N, D)

    hs = _run_l1(x, s1, cnt, root1, W1, b1.reshape(1, D))

    s2 = _scatter_fn(hs.reshape(NSL * N, SL), src, dst, et)
    s2 = s2.reshape(R, N, D)

    return _run_l2(hs, s2, cnt, root2, W2, b2.reshape(1, D), batch3)


# speedup vs baseline: 1.1514x; 1.1514x over previous
import jax
import jax.numpy as jnp
from jax import lax
from jax.experimental import pallas as pl
from jax.experimental.pallas import tpu as pltpu
from jax.experimental.pallas import tpu_sc as plsc

N = 10000
E = 320000
D = 128
R = 6
G = 128
NSL = 8
SL = D // NSL
NK = R * N
NC = 2
NS = 16
LANES = 16
PPC = NSL // NC

_CB = 2000
_C_BLOCKS = (E // (NC * NS)) // _CB
_C_STRIPE = NK // NS


def _counts_body(eg_h, out_h, keyb0, keyb1, ones_v, zb, cacc, se0, se1):
    c = lax.axis_index("c")
    s = lax.axis_index("s")

    zeros16 = jnp.zeros((LANES,), jnp.float32)
    ones16 = jnp.ones((LANES,), jnp.float32)

    cbase = c * (E // 2) + s * (E // (NC * NS))

    def _start_k(b, keyb, sem):
        base = pl.multiple_of(cbase + b * _CB, 8)
        return pltpu.async_copy(eg_h.at[1].at[pl.ds(base, _CB)], keyb, sem)

    def _wait_k(keyb, sem):
        pltpu.make_async_copy(eg_h.at[1].at[pl.ds(0, _CB)], keyb, sem).wait()

    _start_k(0, keyb0, se0)

    def _fill(i, _):
        zb[i, :] = zeros16
        return ()
    lax.fori_loop(0, zb.shape[0], _fill, ())

    def _fill1(i, _):
        ones_v[i, :] = ones16
        return ()
    lax.fori_loop(0, _CB, _fill1, ())

    for k in range(_C_STRIPE // zb.shape[0]):
        pltpu.sync_copy(zb, cacc.at[pl.ds(s * _C_STRIPE + k * zb.shape[0],
                                          zb.shape[0])])
    plsc.subcore_barrier()

    for b in range(_C_BLOCKS):
        kb, sem = (keyb0, se0) if b % 2 == 0 else (keyb1, se1)
        kn, semn = (keyb1, se1) if b % 2 == 0 else (keyb0, se0)
        _wait_k(kb, sem)
        if b + 1 < _C_BLOCKS:
            _start_k(b + 1, kn, semn)
        pltpu.sync_copy(ones_v, cacc.at[kb], add=True)

    plsc.subcore_barrier()

    @pl.when((s == 0) & (c == 0))
    def _():
        pltpu.sync_copy(cacc, out_h.at[0])

    @pl.when((s == 0) & (c == 1))
    def _():
        pltpu.sync_copy(cacc, out_h.at[1])


_counts_fn = pl.kernel(
    _counts_body,
    out_type=jax.ShapeDtypeStruct((NC, NK, LANES), jnp.float32),
    mesh=plsc.VectorSubcoreMesh(core_axis_name="c", subcore_axis_name="s"),
    compiler_params=pltpu.CompilerParams(use_tc_tiling_on_sc=False),
    scratch_types=[
        pltpu.VMEM((_CB,), jnp.int32),
        pltpu.VMEM((_CB,), jnp.int32),
        pltpu.VMEM((_CB, LANES), jnp.float32),
        pltpu.VMEM((750, LANES), jnp.float32),
        pltpu.VMEM_SHARED((NK, LANES), jnp.float32),
        pltpu.SemaphoreType.DMA,
        pltpu.SemaphoreType.DMA,
    ],
)


_EB = 800
_S_BLOCKS = (E // NS) // _EB
_S_PAIRS = (_S_BLOCKS - 1) // 2
_S_STRIPE = NK // NS


def _scatter_body(xs_h, eg_h, out_h,
                  ebuf0, ebuf1, idxb0, keyb0, idxb1, keyb1,
                  rows0, rows1, zb, acc, se0, se1, sg0, sg1):
    c = lax.axis_index("c")
    s = lax.axis_index("s")

    zeros16 = jnp.zeros((LANES,), jnp.float32)

    def _fill(i, _):
        zb[i, :] = zeros16
        return ()
    lax.fori_loop(0, zb.shape[0], _fill, ())

    ebase = s * (E // NS)

    def _start_e(b, ebuf, sem):
        base = pl.multiple_of(ebase + b * _EB, 8)
        return pltpu.async_copy(eg_h.at[:, pl.ds(base, _EB)], ebuf, sem)

    def _wait_e(ebuf, sem):
        pltpu.make_async_copy(eg_h.at[:, pl.ds(0, _EB)], ebuf, sem).wait()

    def _start_g(idxb, rows, sem):
        return pltpu.async_copy(xs_h.at[idxb], rows, sem)

    def _wait_g(idxb, rows, sem):
        pltpu.make_async_copy(xs_h.at[idxb], rows, sem).wait()

    for p_local in range(PPC):
        p = PPC * c + p_local
        col = p * SL
        poff = p * N

        _start_e(0, ebuf0, se0)

        for k in range(_S_STRIPE // zb.shape[0]):
            pltpu.sync_copy(zb, acc.at[pl.ds(s * _S_STRIPE + k * zb.shape[0],
                                             zb.shape[0])])
        plsc.subcore_barrier()

        def _compute(ebuf, idxb, keyb):
            def _vec(j, _):
                s16 = ebuf[0, pl.ds(j * LANES, LANES)]
                k16 = ebuf[1, pl.ds(j * LANES, LANES)]
                idxb[pl.ds(j * LANES, LANES)] = s16 + poff
                keyb[pl.ds(j * LANES, LANES)] = k16
                return ()
            lax.fori_loop(0, _EB // LANES, _vec, ())

        _wait_e(ebuf0, se0)
        _compute(ebuf0, idxb0, keyb0)
        _start_e(1, ebuf1, se1)
        _start_g(idxb0, rows0, sg0)

        def _pair(k, _):
            _wait_e(ebuf1, se1)
            _compute(ebuf1, idxb1, keyb1)
            _start_e(2 * k + 2, ebuf0, se0)
            _wait_g(idxb0, rows0, sg0)
            _start_g(idxb1, rows1, sg1)
            pltpu.sync_copy(rows0, acc.at[keyb0], add=True)
            _wait_e(ebuf0, se0)
            _compute(ebuf0, idxb0, keyb0)

            @pl.when(k < _S_PAIRS - 1)
            def _():
                _start_e(2 * k + 3, ebuf1, se1)

            _wait_g(idxb1, rows1, sg1)
            _start_g(idxb0, rows0, sg0)
            pltpu.sync_copy(rows1, acc.at[keyb1], add=True)
            return ()
        lax.fori_loop(0, _S_PAIRS, _pair, ())

        _wait_g(idxb0, rows0, sg0)
        pltpu.sync_copy(rows0, acc.at[keyb0], add=True)

        plsc.subcore_barrier()

        @pl.when(s == 0)
        def _():
            pltpu.sync_copy(acc, out_h.at[:, pl.ds(col, SL)])

        plsc.subcore_barrier()


_scatter_fn = pl.kernel(
    _scatter_body,
    out_type=jax.ShapeDtypeStruct((NK, D), jnp.float32),
    mesh=plsc.VectorSubcoreMesh(core_axis_name="c", subcore_axis_name="s"),
    compiler_params=pltpu.CompilerParams(use_tc_tiling_on_sc=False),
    scratch_types=[
        pltpu.VMEM((2, _EB), jnp.int32),
        pltpu.VMEM((2, _EB), jnp.int32),
        pltpu.VMEM((_EB,), jnp.int32),
        pltpu.VMEM((_EB,), jnp.int32),
        pltpu.VMEM((_EB,), jnp.int32),
        pltpu.VMEM((_EB,), jnp.int32),
        pltpu.VMEM((_EB, SL), jnp.float32),
        pltpu.VMEM((_EB, SL), jnp.float32),
        pltpu.VMEM((375, SL), jnp.float32),
        pltpu.VMEM_SHARED((NK, SL), jnp.float32),
        pltpu.SemaphoreType.DMA,
        pltpu.SemaphoreType.DMA,
        pltpu.SemaphoreType.DMA,
        pltpu.SemaphoreType.DMA,
    ],
)


_BN = 1000


def _layer_math(xb, s_ref, c_ref, root_ref, w_ref, b_ref):
    acc = jnp.dot(xb, root_ref[...], preferred_element_type=jnp.float32)
    acc = acc + b_ref[...]
    for r in range(R):
        cnt = c_ref[0, r][:, 0:1] + c_ref[1, r][:, 0:1]
        inv = 1.0 / jnp.maximum(cnt, 1.0)
        acc = acc + jnp.dot(s_ref[r] * inv, w_ref[r],
                            preferred_element_type=jnp.float32)
    return jnp.maximum(acc, 0.0)


def _l1_body(x_ref, s_ref, c_ref, root_ref, w_ref, b_ref, o_ref):
    h = _layer_math(x_ref[...], s_ref, c_ref, root_ref, w_ref, b_ref)
    o_ref[...] = jnp.stack([h[:, q * SL:(q + 1) * SL] for q in range(NSL)],
                           axis=0)


def _l2_body(h_ref, s_ref, c_ref, root_ref, w_ref, b_ref, batch_ref, o_ref,
             sum_s, cnt_s):
    i = pl.program_id(0)
    xb = jnp.concatenate([h_ref[q] for q in range(NSL)], axis=-1)
    h2 = _layer_math(xb, s_ref, c_ref, root_ref, w_ref, b_ref)

    bids = batch_ref[0]
    p_oh = (bids == lax.broadcasted_iota(jnp.int32, (G, _BN), 0))
    p_oh = p_oh.astype(jnp.float32)

    @pl.when(i == 0)
    def _():
        sum_s[...] = jnp.zeros_like(sum_s)
        cnt_s[...] = jnp.zeros_like(cnt_s)

    sum_s[...] += jnp.dot(p_oh, h2, preferred_element_type=jnp.float32)
    cnt_s[...] += jnp.sum(p_oh, axis=1, keepdims=True)

    @pl.when(i == pl.num_programs(0) - 1)
    def _():
        o_ref[...] = sum_s[...] / jnp.maximum(cnt_s[...], 1.0)


def _run_l1(x, s1, cnt, root1, w1, b1):
    return pl.pallas_call(
        _l1_body,
        grid=(N // _BN,),
        in_specs=[
            pl.BlockSpec((_BN, D), lambda i: (i, 0)),
            pl.BlockSpec((R, _BN, D), lambda i: (0, i, 0)),
            pl.BlockSpec((NC, R, _BN, LANES), lambda i: (0, 0, i, 0)),
            pl.BlockSpec((D, D), lambda i: (0, 0)),
            pl.BlockSpec((R, D, D), lambda i: (0, 0, 0)),
            pl.BlockSpec((1, D), lambda i: (0, 0)),
        ],
        out_specs=pl.BlockSpec((NSL, _BN, SL), lambda i: (0, i, 0)),
        out_shape=jax.ShapeDtypeStruct((NSL, N, SL), jnp.float32),
    )(x, s1, cnt, root1, w1, b1)


def _run_l2(hs, s2, cnt, root2, w2, b2, batch3):
    return pl.pallas_call(
        _l2_body,
        grid=(N // _BN,),
        in_specs=[
            pl.BlockSpec((NSL, _BN, SL), lambda i: (0, i, 0)),
            pl.BlockSpec((R, _BN, D), lambda i: (0, i, 0)),
            pl.BlockSpec((NC, R, _BN, LANES), lambda i: (0, 0, i, 0)),
            pl.BlockSpec((D, D), lambda i: (0, 0)),
            pl.BlockSpec((R, D, D), lambda i: (0, 0, 0)),
            pl.BlockSpec((1, D), lambda i: (0, 0)),
            pl.BlockSpec((1, 1, _BN), lambda i: (i, 0, 0)),
        ],
        out_specs=pl.BlockSpec((G, D), lambda i: (0, 0)),
        out_shape=jax.ShapeDtypeStruct((G, D), jnp.float32),
        scratch_shapes=[
            pltpu.VMEM((G, D), jnp.float32),
            pltpu.VMEM((G, 1), jnp.float32),
        ],
    )(hs, s2, cnt, root2, w2, b2, batch3)


def kernel(x, edge_index, edge_type, batch, W1, root1, b1, W2, root2, b2):
    src = edge_index[0].astype(jnp.int32)
    dst = edge_index[1].astype(jnp.int32)
    et = edge_type.astype(jnp.int32)
    batch3 = batch.astype(jnp.int32).reshape(N // _BN, 1, _BN)

    xs = x.reshape(N, NSL, SL).transpose(1, 0, 2).reshape(NSL * N, SL)
    edges = jnp.stack([src, et * N + dst])

    cnt = _counts_fn(edges)
    cnt = cnt.reshape(NC, R, N, LANES)

    s1 = _scatter_fn(xs, edges)
    s1 = s1.reshape(R, N, D)

    hs = _run_l1(x, s1, cnt, root1, W1, b1.reshape(1, D))

    s2 = _scatter_fn(hs.reshape(NSL * N, SL), edges)
    s2 = s2.reshape(R, N, D)

    return _run_l2(hs, s2, cnt, root2, W2, b2.reshape(1, D), batch3)


# speedup vs baseline: 1.2316x; 1.0696x over previous
import jax
import jax.numpy as jnp
from jax import lax
from jax.experimental import pallas as pl
from jax.experimental.pallas import tpu as pltpu
from jax.experimental.pallas import tpu_sc as plsc

N = 10000
E = 320000
D = 128
R = 6
G = 128
NSL = 8
SL = D // NSL
NK = R * N
NC = 2
NS = 16
LANES = 16
PPC = NSL // NC

_CB = 2000
_C_BLOCKS = (E // (NC * NS)) // _CB
_C_STRIPE = NK // NS


def _counts_body(eg_h, out_h, keyb0, keyb1, ones_v, zb, cacc, se0, se1):
    c = lax.axis_index("c")
    s = lax.axis_index("s")

    zeros16 = jnp.zeros((LANES,), jnp.float32)
    ones16 = jnp.ones((LANES,), jnp.float32)

    cbase = c * (E // 2) + s * (E // (NC * NS))

    def _start_k(b, keyb, sem):
        base = pl.multiple_of(cbase + b * _CB, 8)
        return pltpu.async_copy(eg_h.at[1].at[pl.ds(base, _CB)], keyb, sem)

    def _wait_k(keyb, sem):
        pltpu.make_async_copy(eg_h.at[1].at[pl.ds(0, _CB)], keyb, sem).wait()

    _start_k(0, keyb0, se0)

    def _fill(i, _):
        zb[i, :] = zeros16
        return ()
    lax.fori_loop(0, zb.shape[0], _fill, ())

    def _fill1(i, _):
        ones_v[i, :] = ones16
        return ()
    lax.fori_loop(0, _CB, _fill1, ())

    for k in range(_C_STRIPE // zb.shape[0]):
        pltpu.sync_copy(zb, cacc.at[pl.ds(s * _C_STRIPE + k * zb.shape[0],
                                          zb.shape[0])])
    plsc.subcore_barrier()

    for b in range(_C_BLOCKS):
        kb, sem = (keyb0, se0) if b % 2 == 0 else (keyb1, se1)
        kn, semn = (keyb1, se1) if b % 2 == 0 else (keyb0, se0)
        _wait_k(kb, sem)
        if b + 1 < _C_BLOCKS:
            _start_k(b + 1, kn, semn)
        pltpu.sync_copy(ones_v, cacc.at[kb], add=True)

    plsc.subcore_barrier()

    @pl.when((s == 0) & (c == 0))
    def _():
        pltpu.sync_copy(cacc, out_h.at[0])

    @pl.when((s == 0) & (c == 1))
    def _():
        pltpu.sync_copy(cacc, out_h.at[1])


_counts_fn = pl.kernel(
    _counts_body,
    out_type=jax.ShapeDtypeStruct((NC, NK, LANES), jnp.float32),
    mesh=plsc.VectorSubcoreMesh(core_axis_name="c", subcore_axis_name="s"),
    compiler_params=pltpu.CompilerParams(use_tc_tiling_on_sc=False),
    scratch_types=[
        pltpu.VMEM((_CB,), jnp.int32),
        pltpu.VMEM((_CB,), jnp.int32),
        pltpu.VMEM((_CB, LANES), jnp.float32),
        pltpu.VMEM((750, LANES), jnp.float32),
        pltpu.VMEM_SHARED((NK, LANES), jnp.float32),
        pltpu.SemaphoreType.DMA,
        pltpu.SemaphoreType.DMA,
    ],
)


_EB = 800
_S_BLOCKS = (E // NS) // _EB
_S_PAIRS = (_S_BLOCKS - 1) // 2
_S_STRIPE = NK // NS


def _scatter_body(xs_h, eg_h, out_h,
                  ebuf0, ebuf1, idxb0, keyb0, idxb1, keyb1,
                  rows0, rows1, zb, acc, xsl, se0, se1, sg0, sg1):
    c = lax.axis_index("c")
    s = lax.axis_index("s")

    zeros16 = jnp.zeros((LANES,), jnp.float32)

    def _fill(i, _):
        zb[i, :] = zeros16
        return ()
    lax.fori_loop(0, zb.shape[0], _fill, ())

    ebase = s * (E // NS)

    def _start_e(b, ebuf, sem):
        base = pl.multiple_of(ebase + b * _EB, 8)
        return pltpu.async_copy(eg_h.at[:, pl.ds(base, _EB)], ebuf, sem)

    def _wait_e(ebuf, sem):
        pltpu.make_async_copy(eg_h.at[:, pl.ds(0, _EB)], ebuf, sem).wait()

    def _start_g(idxb, rows, sem):
        return pltpu.async_copy(xsl.at[idxb], rows, sem)

    def _wait_g(idxb, rows, sem):
        pltpu.make_async_copy(xsl.at[idxb], rows, sem).wait()

    for p_local in range(PPC):
        p = PPC * c + p_local
        col = p * SL

        _start_e(0, ebuf0, se0)

        @pl.when(s == 0)
        def _():
            pltpu.sync_copy(xs_h.at[pl.ds(p * N, N)], xsl)

        for k in range(_S_STRIPE // zb.shape[0]):
            pltpu.sync_copy(zb, acc.at[pl.ds(s * _S_STRIPE + k * zb.shape[0],
                                             zb.shape[0])])
        plsc.subcore_barrier()

        def _compute(ebuf, idxb, keyb):
            def _vec(j, _):
                s16 = ebuf[0, pl.ds(j * LANES, LANES)]
                k16 = ebuf[1, pl.ds(j * LANES, LANES)]
                idxb[pl.ds(j * LANES, LANES)] = s16
                keyb[pl.ds(j * LANES, LANES)] = k16
                return ()
            lax.fori_loop(0, _EB // LANES, _vec, ())

        _wait_e(ebuf0, se0)
        _compute(ebuf0, idxb0, keyb0)
        _start_e(1, ebuf1, se1)
        _start_g(idxb0, rows0, sg0)

        def _pair(k, _):
            _wait_e(ebuf1, se1)
            _compute(ebuf1, idxb1, keyb1)
            _start_e(2 * k + 2, ebuf0, se0)
            _wait_g(idxb0, rows0, sg0)
            _start_g(idxb1, rows1, sg1)
            pltpu.sync_copy(rows0, acc.at[keyb0], add=True)
            _wait_e(ebuf0, se0)
            _compute(ebuf0, idxb0, keyb0)

            @pl.when(k < _S_PAIRS - 1)
            def _():
                _start_e(2 * k + 3, ebuf1, se1)

            _wait_g(idxb1, rows1, sg1)
            _start_g(idxb0, rows0, sg0)
            pltpu.sync_copy(rows1, acc.at[keyb1], add=True)
            return ()
        lax.fori_loop(0, _S_PAIRS, _pair, ())

        _wait_g(idxb0, rows0, sg0)
        pltpu.sync_copy(rows0, acc.at[keyb0], add=True)

        plsc.subcore_barrier()

        @pl.when(s == 0)
        def _():
            pltpu.sync_copy(acc, out_h.at[:, pl.ds(col, SL)])

        plsc.subcore_barrier()


_scatter_fn = pl.kernel(
    _scatter_body,
    out_type=jax.ShapeDtypeStruct((NK, D), jnp.float32),
    mesh=plsc.VectorSubcoreMesh(core_axis_name="c", subcore_axis_name="s"),
    compiler_params=pltpu.CompilerParams(use_tc_tiling_on_sc=False),
    scratch_types=[
        pltpu.VMEM((2, _EB), jnp.int32),
        pltpu.VMEM((2, _EB), jnp.int32),
        pltpu.VMEM((_EB,), jnp.int32),
        pltpu.VMEM((_EB,), jnp.int32),
        pltpu.VMEM((_EB,), jnp.int32),
        pltpu.VMEM((_EB,), jnp.int32),
        pltpu.VMEM((_EB, SL), jnp.float32),
        pltpu.VMEM((_EB, SL), jnp.float32),
        pltpu.VMEM((375, SL), jnp.float32),
        pltpu.VMEM_SHARED((NK, SL), jnp.float32),
        pltpu.VMEM_SHARED((N, SL), jnp.float32),
        pltpu.SemaphoreType.DMA,
        pltpu.SemaphoreType.DMA,
        pltpu.SemaphoreType.DMA,
        pltpu.SemaphoreType.DMA,
    ],
)


_BN = 1000


def _layer_math(xb, s_ref, c_ref, root_ref, w_ref, b_ref):
    acc = jnp.dot(xb, root_ref[...], preferred_element_type=jnp.float32)
    acc = acc + b_ref[...]
    for r in range(R):
        cnt = c_ref[0, r][:, 0:1] + c_ref[1, r][:, 0:1]
        inv = 1.0 / jnp.maximum(cnt, 1.0)
        acc = acc + jnp.dot(s_ref[r] * inv, w_ref[r],
                            preferred_element_type=jnp.float32)
    return jnp.maximum(acc, 0.0)


def _l1_body(x_ref, s_ref, c_ref, root_ref, w_ref, b_ref, o_ref):
    h = _layer_math(x_ref[...], s_ref, c_ref, root_ref, w_ref, b_ref)
    o_ref[...] = jnp.stack([h[:, q * SL:(q + 1) * SL] for q in range(NSL)],
                           axis=0)


def _l2_body(h_ref, s_ref, c_ref, root_ref, w_ref, b_ref, batch_ref, o_ref,
             sum_s, cnt_s):
    i = pl.program_id(0)
    xb = jnp.concatenate([h_ref[q] for q in range(NSL)], axis=-1)
    h2 = _layer_math(xb, s_ref, c_ref, root_ref, w_ref, b_ref)

    bids = batch_ref[0]
    p_oh = (bids == lax.broadcasted_iota(jnp.int32, (G, _BN), 0))
    p_oh = p_oh.astype(jnp.float32)

    @pl.when(i == 0)
    def _():
        sum_s[...] = jnp.zeros_like(sum_s)
        cnt_s[...] = jnp.zeros_like(cnt_s)

    sum_s[...] += jnp.dot(p_oh, h2, preferred_element_type=jnp.float32)
    cnt_s[...] += jnp.sum(p_oh, axis=1, keepdims=True)

    @pl.when(i == pl.num_programs(0) - 1)
    def _():
        o_ref[...] = sum_s[...] / jnp.maximum(cnt_s[...], 1.0)


def _run_l1(x, s1, cnt, root1, w1, b1):
    return pl.pallas_call(
        _l1_body,
        grid=(N // _BN,),
        in_specs=[
            pl.BlockSpec((_BN, D), lambda i: (i, 0)),
            pl.BlockSpec((R, _BN, D), lambda i: (0, i, 0)),
            pl.BlockSpec((NC, R, _BN, LANES), lambda i: (0, 0, i, 0)),
            pl.BlockSpec((D, D), lambda i: (0, 0)),
            pl.BlockSpec((R, D, D), lambda i: (0, 0, 0)),
            pl.BlockSpec((1, D), lambda i: (0, 0)),
        ],
        out_specs=pl.BlockSpec((NSL, _BN, SL), lambda i: (0, i, 0)),
        out_shape=jax.ShapeDtypeStruct((NSL, N, SL), jnp.float32),
    )(x, s1, cnt, root1, w1, b1)


def _run_l2(hs, s2, cnt, root2, w2, b2, batch3):
    return pl.pallas_call(
        _l2_body,
        grid=(N // _BN,),
        in_specs=[
            pl.BlockSpec((NSL, _BN, SL), lambda i: (0, i, 0)),
            pl.BlockSpec((R, _BN, D), lambda i: (0, i, 0)),
            pl.BlockSpec((NC, R, _BN, LANES), lambda i: (0, 0, i, 0)),
            pl.BlockSpec((D, D), lambda i: (0, 0)),
            pl.BlockSpec((R, D, D), lambda i: (0, 0, 0)),
            pl.BlockSpec((1, D), lambda i: (0, 0)),
            pl.BlockSpec((1, 1, _BN), lambda i: (i, 0, 0)),
        ],
        out_specs=pl.BlockSpec((G, D), lambda i: (0, 0)),
        out_shape=jax.ShapeDtypeStruct((G, D), jnp.float32),
        scratch_shapes=[
            pltpu.VMEM((G, D), jnp.float32),
            pltpu.VMEM((G, 1), jnp.float32),
        ],
    )(hs, s2, cnt, root2, w2, b2, batch3)


def kernel(x, edge_index, edge_type, batch, W1, root1, b1, W2, root2, b2):
    src = edge_index[0].astype(jnp.int32)
    dst = edge_index[1].astype(jnp.int32)
    et = edge_type.astype(jnp.int32)
    batch3 = batch.astype(jnp.int32).reshape(N // _BN, 1, _BN)

    xs = x.reshape(N, NSL, SL).transpose(1, 0, 2).reshape(NSL * N, SL)
    edges = jnp.stack([src, et * N + dst])

    cnt = _counts_fn(edges)
    cnt = cnt.reshape(NC, R, N, LANES)

    s1 = _scatter_fn(xs, edges)
    s1 = s1.reshape(R, N, D)

    hs = _run_l1(x, s1, cnt, root1, W1, b1.reshape(1, D))

    s2 = _scatter_fn(hs.reshape(NSL * N, SL), edges)
    s2 = s2.reshape(R, N, D)

    return _run_l2(hs, s2, cnt, root2, W2, b2.reshape(1, D), batch3)


# speedup vs baseline: 1.2329x; 1.0010x over previous
import jax
import jax.numpy as jnp
from jax import lax
from jax.experimental import pallas as pl
from jax.experimental.pallas import tpu as pltpu
from jax.experimental.pallas import tpu_sc as plsc

N = 10000
E = 320000
D = 128
R = 6
G = 128
NSL = 8
SL = D // NSL
NK = R * N
NC = 2
NS = 16
LANES = 16
PPC = NSL // NC

_CB = 2000
_C_BLOCKS = (E // (NC * NS)) // _CB
_C_STRIPE = NK // NS


def _counts_body(eg_h, out_h, keyb0, keyb1, ones_v, zb, cacc, se0, se1):
    c = lax.axis_index("c")
    s = lax.axis_index("s")

    zeros16 = jnp.zeros((LANES,), jnp.float32)
    ones16 = jnp.ones((LANES,), jnp.float32)

    cbase = c * (E // 2) + s * (E // (NC * NS))

    def _start_k(b, keyb, sem):
        base = pl.multiple_of(cbase + b * _CB, 8)
        return pltpu.async_copy(eg_h.at[1].at[pl.ds(base, _CB)], keyb, sem)

    def _wait_k(keyb, sem):
        pltpu.make_async_copy(eg_h.at[1].at[pl.ds(0, _CB)], keyb, sem).wait()

    _start_k(0, keyb0, se0)

    def _fill(i, _):
        zb[i, :] = zeros16
        return ()
    lax.fori_loop(0, zb.shape[0], _fill, ())

    def _fill1(i, _):
        ones_v[i, :] = ones16
        return ()
    lax.fori_loop(0, _CB, _fill1, ())

    for k in range(_C_STRIPE // zb.shape[0]):
        pltpu.sync_copy(zb, cacc.at[pl.ds(s * _C_STRIPE + k * zb.shape[0],
                                          zb.shape[0])])
    plsc.subcore_barrier()

    for b in range(_C_BLOCKS):
        kb, sem = (keyb0, se0) if b % 2 == 0 else (keyb1, se1)
        kn, semn = (keyb1, se1) if b % 2 == 0 else (keyb0, se0)
        _wait_k(kb, sem)
        if b + 1 < _C_BLOCKS:
            _start_k(b + 1, kn, semn)
        pltpu.sync_copy(ones_v, cacc.at[kb], add=True)

    plsc.subcore_barrier()

    @pl.when((s == 0) & (c == 0))
    def _():
        pltpu.sync_copy(cacc, out_h.at[0])

    @pl.when((s == 0) & (c == 1))
    def _():
        pltpu.sync_copy(cacc, out_h.at[1])


_counts_fn = pl.kernel(
    _counts_body,
    out_type=jax.ShapeDtypeStruct((NC, NK, LANES), jnp.float32),
    mesh=plsc.VectorSubcoreMesh(core_axis_name="c", subcore_axis_name="s"),
    compiler_params=pltpu.CompilerParams(use_tc_tiling_on_sc=False),
    scratch_types=[
        pltpu.VMEM((_CB,), jnp.int32),
        pltpu.VMEM((_CB,), jnp.int32),
        pltpu.VMEM((_CB, LANES), jnp.float32),
        pltpu.VMEM((750, LANES), jnp.float32),
        pltpu.VMEM_SHARED((NK, LANES), jnp.float32),
        pltpu.SemaphoreType.DMA,
        pltpu.SemaphoreType.DMA,
    ],
)


_EB = 800
_S_BLOCKS = (E // NS) // _EB
_S_PAIRS = (_S_BLOCKS - 1) // 2
_S_STRIPE = NK // NS


def _scatter_body(xs_h, eg_h, out_h,
                  ebuf0, ebuf1, idxb0, keyb0, idxb1, keyb1,
                  rows0, rows1, zb, acc, xsl, se0, se1, sg0, sg1):
    c = lax.axis_index("c")
    s = lax.axis_index("s")

    zeros16 = jnp.zeros((LANES,), jnp.float32)

    def _fill(i, _):
        zb[i, :] = zeros16
        return ()
    lax.fori_loop(0, zb.shape[0], _fill, ())

    ebase = s * (E // NS)

    def _start_e(b, ebuf, sem):
        base = pl.multiple_of(ebase + b * _EB, 8)
        return pltpu.async_copy(eg_h.at[:, pl.ds(base, _EB)], ebuf, sem)

    def _wait_e(ebuf, sem):
        pltpu.make_async_copy(eg_h.at[:, pl.ds(0, _EB)], ebuf, sem).wait()

    def _start_g(idxb, rows, sem):
        return pltpu.async_copy(xsl.at[idxb], rows, sem)

    def _wait_g(idxb, rows, sem):
        pltpu.make_async_copy(xsl.at[idxb], rows, sem).wait()

    def _zero_stripe():
        for k in range(_S_STRIPE // zb.shape[0]):
            pltpu.sync_copy(zb, acc.at[pl.ds(s * _S_STRIPE + k * zb.shape[0],
                                             zb.shape[0])])

    def _stage(p):
        pltpu.sync_copy(xs_h.at[pl.ds(p * N, N)], xsl)

    for p_local in range(PPC):
        p = PPC * c + p_local
        col = p * SL

        _start_e(0, ebuf0, se0)

        if p_local == 0:
            @pl.when(s == 0)
            def _():
                _stage(p)

            _zero_stripe()
        plsc.subcore_barrier()

        def _compute(ebuf, idxb, keyb):
            def _vec(j, _):
                s16 = ebuf[0, pl.ds(j * LANES, LANES)]
                k16 = ebuf[1, pl.ds(j * LANES, LANES)]
                idxb[pl.ds(j * LANES, LANES)] = s16
                keyb[pl.ds(j * LANES, LANES)] = k16
                return ()
            lax.fori_loop(0, _EB // LANES, _vec, ())

        _wait_e(ebuf0, se0)
        _compute(ebuf0, idxb0, keyb0)
        _start_e(1, ebuf1, se1)
        _start_g(idxb0, rows0, sg0)

        def _pair(k, _):
            _wait_e(ebuf1, se1)
            _compute(ebuf1, idxb1, keyb1)
            _start_e(2 * k + 2, ebuf0, se0)
            _wait_g(idxb0, rows0, sg0)
            _start_g(idxb1, rows1, sg1)
            pltpu.sync_copy(rows0, acc.at[keyb0], add=True)
            _wait_e(ebuf0, se0)
            _compute(ebuf0, idxb0, keyb0)

            @pl.when(k < _S_PAIRS - 1)
            def _():
                _start_e(2 * k + 3, ebuf1, se1)

            _wait_g(idxb1, rows1, sg1)
            _start_g(idxb0, rows0, sg0)
            pltpu.sync_copy(rows1, acc.at[keyb1], add=True)
            return ()
        lax.fori_loop(0, _S_PAIRS, _pair, ())

        _wait_g(idxb0, rows0, sg0)
        pltpu.sync_copy(rows0, acc.at[keyb0], add=True)

        plsc.subcore_barrier()

        stripe = pl.ds(s * _S_STRIPE, _S_STRIPE)
        pltpu.sync_copy(acc.at[stripe], out_h.at[stripe, pl.ds(col, SL)])
        if p_local < PPC - 1:
            _zero_stripe()

            @pl.when(s == 0)
            def _():
                _stage(p + 1)


_scatter_fn = pl.kernel(
    _scatter_body,
    out_type=jax.ShapeDtypeStruct((NK, D), jnp.float32),
    mesh=plsc.VectorSubcoreMesh(core_axis_name="c", subcore_axis_name="s"),
    compiler_params=pltpu.CompilerParams(use_tc_tiling_on_sc=False),
    scratch_types=[
        pltpu.VMEM((2, _EB), jnp.int32),
        pltpu.VMEM((2, _EB), jnp.int32),
        pltpu.VMEM((_EB,), jnp.int32),
        pltpu.VMEM((_EB,), jnp.int32),
        pltpu.VMEM((_EB,), jnp.int32),
        pltpu.VMEM((_EB,), jnp.int32),
        pltpu.VMEM((_EB, SL), jnp.float32),
        pltpu.VMEM((_EB, SL), jnp.float32),
        pltpu.VMEM((750, SL), jnp.float32),
        pltpu.VMEM_SHARED((NK, SL), jnp.float32),
        pltpu.VMEM_SHARED((N, SL), jnp.float32),
        pltpu.SemaphoreType.DMA,
        pltpu.SemaphoreType.DMA,
        pltpu.SemaphoreType.DMA,
        pltpu.SemaphoreType.DMA,
    ],
)


_BN = 1000


def _layer_math(xb, s_ref, c_ref, root_ref, w_ref, b_ref):
    acc = jnp.dot(xb, root_ref[...], preferred_element_type=jnp.float32)
    acc = acc + b_ref[...]
    for r in range(R):
        cnt = c_ref[0, r][:, 0:1] + c_ref[1, r][:, 0:1]
        inv = 1.0 / jnp.maximum(cnt, 1.0)
        acc = acc + jnp.dot(s_ref[r] * inv, w_ref[r],
                            preferred_element_type=jnp.float32)
    return jnp.maximum(acc, 0.0)


def _l1_body(x_ref, s_ref, c_ref, root_ref, w_ref, b_ref, o_ref):
    h = _layer_math(x_ref[...], s_ref, c_ref, root_ref, w_ref, b_ref)
    o_ref[...] = jnp.stack([h[:, q * SL:(q + 1) * SL] for q in range(NSL)],
                           axis=0)


def _l2_body(h_ref, s_ref, c_ref, root_ref, w_ref, b_ref, batch_ref, o_ref,
             sum_s, cnt_s):
    i = pl.program_id(0)
    xb = jnp.concatenate([h_ref[q] for q in range(NSL)], axis=-1)
    h2 = _layer_math(xb, s_ref, c_ref, root_ref, w_ref, b_ref)

    bids = batch_ref[0]
    p_oh = (bids == lax.broadcasted_iota(jnp.int32, (G, _BN), 0))
    p_oh = p_oh.astype(jnp.float32)

    @pl.when(i == 0)
    def _():
        sum_s[...] = jnp.zeros_like(sum_s)
        cnt_s[...] = jnp.zeros_like(cnt_s)

    sum_s[...] += jnp.dot(p_oh, h2, preferred_element_type=jnp.float32)
    cnt_s[...] += jnp.sum(p_oh, axis=1, keepdims=True)

    @pl.when(i == pl.num_programs(0) - 1)
    def _():
        o_ref[...] = sum_s[...] / jnp.maximum(cnt_s[...], 1.0)


def _run_l1(x, s1, cnt, root1, w1, b1):
    return pl.pallas_call(
        _l1_body,
        grid=(N // _BN,),
        in_specs=[
            pl.BlockSpec((_BN, D), lambda i: (i, 0)),
            pl.BlockSpec((R, _BN, D), lambda i: (0, i, 0)),
            pl.BlockSpec((NC, R, _BN, LANES), lambda i: (0, 0, i, 0)),
            pl.BlockSpec((D, D), lambda i: (0, 0)),
            pl.BlockSpec((R, D, D), lambda i: (0, 0, 0)),
            pl.BlockSpec((1, D), lambda i: (0, 0)),
        ],
        out_specs=pl.BlockSpec((NSL, _BN, SL), lambda i: (0, i, 0)),
        out_shape=jax.ShapeDtypeStruct((NSL, N, SL), jnp.float32),
    )(x, s1, cnt, root1, w1, b1)


def _run_l2(hs, s2, cnt, root2, w2, b2, batch3):
    return pl.pallas_call(
        _l2_body,
        grid=(N // _BN,),
        in_specs=[
            pl.BlockSpec((NSL, _BN, SL), lambda i: (0, i, 0)),
            pl.BlockSpec((R, _BN, D), lambda i: (0, i, 0)),
            pl.BlockSpec((NC, R, _BN, LANES), lambda i: (0, 0, i, 0)),
            pl.BlockSpec((D, D), lambda i: (0, 0)),
            pl.BlockSpec((R, D, D), lambda i: (0, 0, 0)),
            pl.BlockSpec((1, D), lambda i: (0, 0)),
            pl.BlockSpec((1, 1, _BN), lambda i: (i, 0, 0)),
        ],
        out_specs=pl.BlockSpec((G, D), lambda i: (0, 0)),
        out_shape=jax.ShapeDtypeStruct((G, D), jnp.float32),
        scratch_shapes=[
            pltpu.VMEM((G, D), jnp.float32),
            pltpu.VMEM((G, 1), jnp.float32),
        ],
    )(hs, s2, cnt, root2, w2, b2, batch3)


def kernel(x, edge_index, edge_type, batch, W1, root1, b1, W2, root2, b2):
    src = edge_index[0].astype(jnp.int32)
    dst = edge_index[1].astype(jnp.int32)
    et = edge_type.astype(jnp.int32)
    batch3 = batch.astype(jnp.int32).reshape(N // _BN, 1, _BN)

    xs = x.reshape(N, NSL, SL).transpose(1, 0, 2).reshape(NSL * N, SL)
    edges = jnp.stack([src, et * N + dst])

    cnt = _counts_fn(edges)
    cnt = cnt.reshape(NC, R, N, LANES)

    s1 = _scatter_fn(xs, edges)
    s1 = s1.reshape(R, N, D)

    hs = _run_l1(x, s1, cnt, root1, W1, b1.reshape(1, D))

    s2 = _scatter_fn(hs.reshape(NSL * N, SL), edges)
    s2 = s2.reshape(R, N, D)

    return _run_l2(hs, s2, cnt, root2, W2, b2.reshape(1, D), batch3)


# speedup vs baseline: 1.3511x; 1.0959x over previous
import jax
import jax.numpy as jnp
from jax import lax
from jax.experimental import pallas as pl
from jax.experimental.pallas import tpu as pltpu
from jax.experimental.pallas import tpu_sc as plsc

N = 10000
E = 320000
D = 128
R = 6
G = 128
NSL = 8
SL = D // NSL
NK = R * N
NC = 2
NS = 16
LANES = 16
PPC = NSL // NC

_CB = 2000
_C_BLOCKS = (E // (NC * NS)) // _CB
_C_STRIPE = NK // NS


def _counts_body(eg_h, out_h, keyb0, keyb1, ones_v, zb, cacc, se0, se1):
    c = lax.axis_index("c")
    s = lax.axis_index("s")

    zeros16 = jnp.zeros((LANES,), jnp.float32)
    ones16 = jnp.ones((LANES,), jnp.float32)

    cbase = c * (E // 2) + s * (E // (NC * NS))

    def _start_k(b, keyb, sem):
        base = pl.multiple_of(cbase + b * _CB, 8)
        return pltpu.async_copy(eg_h.at[1].at[pl.ds(base, _CB)], keyb, sem)

    def _wait_k(keyb, sem):
        pltpu.make_async_copy(eg_h.at[1].at[pl.ds(0, _CB)], keyb, sem).wait()

    _start_k(0, keyb0, se0)

    def _fill(i, _):
        zb[i, :] = zeros16
        return ()
    lax.fori_loop(0, zb.shape[0], _fill, ())

    def _fill1(i, _):
        ones_v[i, :] = ones16
        return ()
    lax.fori_loop(0, _CB, _fill1, ())

    for k in range(_C_STRIPE // zb.shape[0]):
        pltpu.sync_copy(zb, cacc.at[pl.ds(s * _C_STRIPE + k * zb.shape[0],
                                          zb.shape[0])])
    plsc.subcore_barrier()

    for b in range(_C_BLOCKS):
        kb, sem = (keyb0, se0) if b % 2 == 0 else (keyb1, se1)
        kn, semn = (keyb1, se1) if b % 2 == 0 else (keyb0, se0)
        _wait_k(kb, sem)
        if b + 1 < _C_BLOCKS:
            _start_k(b + 1, kn, semn)
        pltpu.sync_copy(ones_v, cacc.at[kb], add=True)

    plsc.subcore_barrier()

    @pl.when((s == 0) & (c == 0))
    def _():
        pltpu.sync_copy(cacc, out_h.at[0])

    @pl.when((s == 0) & (c == 1))
    def _():
        pltpu.sync_copy(cacc, out_h.at[1])


_counts_fn = pl.kernel(
    _counts_body,
    out_type=jax.ShapeDtypeStruct((NC, NK, LANES), jnp.float32),
    mesh=plsc.VectorSubcoreMesh(core_axis_name="c", subcore_axis_name="s"),
    compiler_params=pltpu.CompilerParams(use_tc_tiling_on_sc=False),
    scratch_types=[
        pltpu.VMEM((_CB,), jnp.int32),
        pltpu.VMEM((_CB,), jnp.int32),
        pltpu.VMEM((_CB, LANES), jnp.float32),
        pltpu.VMEM((750, LANES), jnp.float32),
        pltpu.VMEM_SHARED((NK, LANES), jnp.float32),
        pltpu.SemaphoreType.DMA,
        pltpu.SemaphoreType.DMA,
    ],
)


_EB = 800
_S_BLOCKS = (E // NS) // _EB
_S_PAIRS = (_S_BLOCKS - 1) // 2
_S_STRIPE = NK // NS


def _scatter_body(xs_h, eg_h, out_h,
                  ebuf0, ebuf1, idxb0, keyb0, idxb1, keyb1,
                  rows0, rows1, zb, acc, xsl, se0, se1, sg0, sg1):
    c = lax.axis_index("c")
    s = lax.axis_index("s")

    zeros16 = jnp.zeros((LANES,), jnp.float32)

    def _fill(i, _):
        zb[i, :] = zeros16
        return ()
    lax.fori_loop(0, zb.shape[0], _fill, ())

    ebase = s * (E // NS)

    def _start_e(b, ebuf, sem):
        base = pl.multiple_of(ebase + b * _EB, 8)
        return pltpu.async_copy(eg_h.at[:, pl.ds(base, _EB)], ebuf, sem)

    def _wait_e(ebuf, sem):
        pltpu.make_async_copy(eg_h.at[:, pl.ds(0, _EB)], ebuf, sem).wait()

    def _start_g(idxb, rows, sem):
        return pltpu.async_copy(xsl.at[idxb], rows, sem)

    def _wait_g(idxb, rows, sem):
        pltpu.make_async_copy(xsl.at[idxb], rows, sem).wait()

    def _zero_stripe():
        for k in range(_S_STRIPE // zb.shape[0]):
            pltpu.sync_copy(zb, acc.at[pl.ds(s * _S_STRIPE + k * zb.shape[0],
                                             zb.shape[0])])

    def _stage(p):
        pltpu.sync_copy(xs_h.at[:, pl.ds(p * SL, SL)], xsl)

    for p_local in range(PPC):
        p = PPC * c + p_local
        col = p * SL

        _start_e(0, ebuf0, se0)

        if p_local == 0:
            @pl.when(s == 0)
            def _():
                _stage(p)

            _zero_stripe()
        plsc.subcore_barrier()

        def _compute(ebuf, idxb, keyb):
            def _vec(j, _):
                s16 = ebuf[0, pl.ds(j * LANES, LANES)]
                k16 = ebuf[1, pl.ds(j * LANES, LANES)]
                idxb[pl.ds(j * LANES, LANES)] = s16
                keyb[pl.ds(j * LANES, LANES)] = k16
                return ()
            lax.fori_loop(0, _EB // LANES, _vec, ())

        _wait_e(ebuf0, se0)
        _compute(ebuf0, idxb0, keyb0)
        _start_e(1, ebuf1, se1)
        _start_g(idxb0, rows0, sg0)

        def _pair(k, _):
            _wait_e(ebuf1, se1)
            _compute(ebuf1, idxb1, keyb1)
            _start_e(2 * k + 2, ebuf0, se0)
            _wait_g(idxb0, rows0, sg0)
            _start_g(idxb1, rows1, sg1)
            pltpu.sync_copy(rows0, acc.at[keyb0], add=True)
            _wait_e(ebuf0, se0)
            _compute(ebuf0, idxb0, keyb0)

            @pl.when(k < _S_PAIRS - 1)
            def _():
                _start_e(2 * k + 3, ebuf1, se1)

            _wait_g(idxb1, rows1, sg1)
            _start_g(idxb0, rows0, sg0)
            pltpu.sync_copy(rows1, acc.at[keyb1], add=True)
            return ()
        lax.fori_loop(0, _S_PAIRS, _pair, ())

        _wait_g(idxb0, rows0, sg0)
        pltpu.sync_copy(rows0, acc.at[keyb0], add=True)

        plsc.subcore_barrier()

        stripe = pl.ds(s * _S_STRIPE, _S_STRIPE)
        pltpu.sync_copy(acc.at[stripe], out_h.at[stripe, pl.ds(col, SL)])
        if p_local < PPC - 1:
            _zero_stripe()

            @pl.when(s == 0)
            def _():
                _stage(p + 1)


_scatter_fn = pl.kernel(
    _scatter_body,
    out_type=jax.ShapeDtypeStruct((NK, D), jnp.float32),
    mesh=plsc.VectorSubcoreMesh(core_axis_name="c", subcore_axis_name="s"),
    compiler_params=pltpu.CompilerParams(use_tc_tiling_on_sc=False),
    scratch_types=[
        pltpu.VMEM((2, _EB), jnp.int32),
        pltpu.VMEM((2, _EB), jnp.int32),
        pltpu.VMEM((_EB,), jnp.int32),
        pltpu.VMEM((_EB,), jnp.int32),
        pltpu.VMEM((_EB,), jnp.int32),
        pltpu.VMEM((_EB,), jnp.int32),
        pltpu.VMEM((_EB, SL), jnp.float32),
        pltpu.VMEM((_EB, SL), jnp.float32),
        pltpu.VMEM((750, SL), jnp.float32),
        pltpu.VMEM_SHARED((NK, SL), jnp.float32),
        pltpu.VMEM_SHARED((N, SL), jnp.float32),
        pltpu.SemaphoreType.DMA,
        pltpu.SemaphoreType.DMA,
        pltpu.SemaphoreType.DMA,
        pltpu.SemaphoreType.DMA,
    ],
)


_BN = 1000


def _layer_math(xb, s_ref, c_ref, root_ref, w_ref, b_ref):
    acc = jnp.dot(xb, root_ref[...], preferred_element_type=jnp.float32)
    acc = acc + b_ref[...]
    for r in range(R):
        cnt = c_ref[0, r][:, 0:1] + c_ref[1, r][:, 0:1]
        inv = 1.0 / jnp.maximum(cnt, 1.0)
        acc = acc + jnp.dot(s_ref[r] * inv, w_ref[r],
                            preferred_element_type=jnp.float32)
    return jnp.maximum(acc, 0.0)


def _l1_body(x_ref, s_ref, c_ref, root_ref, w_ref, b_ref, o_ref):
    o_ref[...] = _layer_math(x_ref[...], s_ref, c_ref, root_ref, w_ref, b_ref)


def _l2_body(h_ref, s_ref, c_ref, root_ref, w_ref, b_ref, batch_ref, o_ref,
             sum_s, cnt_s):
    i = pl.program_id(0)
    h2 = _layer_math(h_ref[...], s_ref, c_ref, root_ref, w_ref, b_ref)

    bids = batch_ref[0]
    p_oh = (bids == lax.broadcasted_iota(jnp.int32, (G, _BN), 0))
    p_oh = p_oh.astype(jnp.float32)

    @pl.when(i == 0)
    def _():
        sum_s[...] = jnp.zeros_like(sum_s)
        cnt_s[...] = jnp.zeros_like(cnt_s)

    sum_s[...] += jnp.dot(p_oh, h2, preferred_element_type=jnp.float32)
    cnt_s[...] += jnp.sum(p_oh, axis=1, keepdims=True)

    @pl.when(i == pl.num_programs(0) - 1)
    def _():
        o_ref[...] = sum_s[...] / jnp.maximum(cnt_s[...], 1.0)


def _run_l1(x, s1, cnt, root1, w1, b1):
    return pl.pallas_call(
        _l1_body,
        grid=(N // _BN,),
        in_specs=[
            pl.BlockSpec((_BN, D), lambda i: (i, 0)),
            pl.BlockSpec((R, _BN, D), lambda i: (0, i, 0)),
            pl.BlockSpec((NC, R, _BN, LANES), lambda i: (0, 0, i, 0)),
            pl.BlockSpec((D, D), lambda i: (0, 0)),
            pl.BlockSpec((R, D, D), lambda i: (0, 0, 0)),
            pl.BlockSpec((1, D), lambda i: (0, 0)),
        ],
        out_specs=pl.BlockSpec((_BN, D), lambda i: (i, 0)),
        out_shape=jax.ShapeDtypeStruct((N, D), jnp.float32),
    )(x, s1, cnt, root1, w1, b1)


def _run_l2(h, s2, cnt, root2, w2, b2, batch3):
    return pl.pallas_call(
        _l2_body,
        grid=(N // _BN,),
        in_specs=[
            pl.BlockSpec((_BN, D), lambda i: (i, 0)),
            pl.BlockSpec((R, _BN, D), lambda i: (0, i, 0)),
            pl.BlockSpec((NC, R, _BN, LANES), lambda i: (0, 0, i, 0)),
            pl.BlockSpec((D, D), lambda i: (0, 0)),
            pl.BlockSpec((R, D, D), lambda i: (0, 0, 0)),
            pl.BlockSpec((1, D), lambda i: (0, 0)),
            pl.BlockSpec((1, 1, _BN), lambda i: (i, 0, 0)),
        ],
        out_specs=pl.BlockSpec((G, D), lambda i: (0, 0)),
        out_shape=jax.ShapeDtypeStruct((G, D), jnp.float32),
        scratch_shapes=[
            pltpu.VMEM((G, D), jnp.float32),
            pltpu.VMEM((G, 1), jnp.float32),
        ],
    )(h, s2, cnt, root2, w2, b2, batch3)


def kernel(x, edge_index, edge_type, batch, W1, root1, b1, W2, root2, b2):
    src = edge_index[0].astype(jnp.int32)
    dst = edge_index[1].astype(jnp.int32)
    et = edge_type.astype(jnp.int32)
    batch3 = batch.astype(jnp.int32).reshape(N // _BN, 1, _BN)

    edges = jnp.stack([src, et * N + dst])

    cnt = _counts_fn(edges)
    cnt = cnt.reshape(NC, R, N, LANES)

    s1 = _scatter_fn(x, edges)
    s1 = s1.reshape(R, N, D)

    h = _run_l1(x, s1, cnt, root1, W1, b1.reshape(1, D))

    s2 = _scatter_fn(h, edges)
    s2 = s2.reshape(R, N, D)

    return _run_l2(h, s2, cnt, root2, W2, b2.reshape(1, D), batch3)


# speedup vs baseline: 1.3512x; 1.0001x over previous
import functools

import jax
import jax.numpy as jnp
from jax import lax
from jax.experimental import pallas as pl
from jax.experimental.pallas import tpu as pltpu
from jax.experimental.pallas import tpu_sc as plsc

N = 10000
E = 320000
D = 128
R = 6
G = 128
NSL = 8
SL = D // NSL
NK = R * N
NC = 2
NS = 16
LANES = 16
PPC = NSL // NC

_EB = 800
_S_BLOCKS = (E // NS) // _EB
_S_PAIRS = (_S_BLOCKS - 1) // 2
_S_STRIPE = NK // NS


def _scatter_body(with_counts, xs_h, eg_h, out_h, cnt_h,
                  ebuf0, ebuf1, idxb0, keyb0, idxb1, keyb1,
                  rows0, rows1, zb, acc, xsl, se0, se1, sg0, sg1):
    c = lax.axis_index("c")
    s = lax.axis_index("s")

    zeros16 = jnp.zeros((LANES,), jnp.float32)

    def _fill(i, _):
        zb[i, :] = zeros16
        return ()
    lax.fori_loop(0, zb.shape[0], _fill, ())

    ebase = s * (E // NS)

    def _start_e(b, ebuf, sem):
        base = pl.multiple_of(ebase + b * _EB, 8)
        return pltpu.async_copy(eg_h.at[:, pl.ds(base, _EB)], ebuf, sem)

    def _wait_e(ebuf, sem):
        pltpu.make_async_copy(eg_h.at[:, pl.ds(0, _EB)], ebuf, sem).wait()

    def _start_g(idxb, rows, sem):
        return pltpu.async_copy(xsl.at[idxb], rows, sem)

    def _wait_g(idxb, rows, sem):
        pltpu.make_async_copy(xsl.at[idxb], rows, sem).wait()

    def _zero_stripe():
        for k in range(_S_STRIPE // zb.shape[0]):
            pltpu.sync_copy(zb, acc.at[pl.ds(s * _S_STRIPE + k * zb.shape[0],
                                             zb.shape[0])])

    def _stage(p):
        pltpu.sync_copy(xs_h.at[:, pl.ds(p * SL, SL)], xsl)

    def _compute(ebuf, idxb, keyb):
        def _vec(j, _):
            s16 = ebuf[0, pl.ds(j * LANES, LANES)]
            k16 = ebuf[1, pl.ds(j * LANES, LANES)]
            idxb[pl.ds(j * LANES, LANES)] = s16
            keyb[pl.ds(j * LANES, LANES)] = k16
            return ()
        lax.fori_loop(0, _EB // LANES, _vec, ())

    stripe = pl.ds(s * _S_STRIPE, _S_STRIPE)

    if with_counts:
        _start_e(0, ebuf0, se0)
        ones16 = jnp.ones((LANES,), jnp.float32)

        def _fill1(i, _):
            rows0[i, :] = ones16
            return ()
        lax.fori_loop(0, _EB, _fill1, ())

        _zero_stripe()

        @pl.when(s == 0)
        def _():
            _stage(PPC * c)

        plsc.subcore_barrier()

        _wait_e(ebuf0, se0)
        _compute(ebuf0, idxb0, keyb0)
        _start_e(1, ebuf1, se1)

        def _cpair(k, _):
            pltpu.sync_copy(rows0, acc.at[keyb0], add=True)
            _wait_e(ebuf1, se1)
            _compute(ebuf1, idxb1, keyb1)
            _start_e(2 * k + 2, ebuf0, se0)
            pltpu.sync_copy(rows0, acc.at[keyb1], add=True)
            _wait_e(ebuf0, se0)
            _compute(ebuf0, idxb0, keyb0)

            @pl.when(k < _S_PAIRS - 1)
            def _():
                _start_e(2 * k + 3, ebuf1, se1)

            return ()
        lax.fori_loop(0, _S_PAIRS, _cpair, ())
        pltpu.sync_copy(rows0, acc.at[keyb0], add=True)

        plsc.subcore_barrier()

        half = pl.ds(c * (NK // 2) + s * (NK // 2 // NS), NK // 2 // NS)
        pltpu.sync_copy(acc.at[half], cnt_h.at[half])

        _zero_stripe()

    for p_local in range(PPC):
        p = PPC * c + p_local
        col = p * SL

        _start_e(0, ebuf0, se0)

        if p_local == 0 and not with_counts:
            @pl.when(s == 0)
            def _():
                _stage(p)

            _zero_stripe()
        plsc.subcore_barrier()

        _wait_e(ebuf0, se0)
        _compute(ebuf0, idxb0, keyb0)
        _start_e(1, ebuf1, se1)
        _start_g(idxb0, rows0, sg0)

        def _pair(k, _):
            _wait_e(ebuf1, se1)
            _compute(ebuf1, idxb1, keyb1)
            _start_e(2 * k + 2, ebuf0, se0)
            _wait_g(idxb0, rows0, sg0)
            _start_g(idxb1, rows1, sg1)
            pltpu.sync_copy(rows0, acc.at[keyb0], add=True)
            _wait_e(ebuf0, se0)
            _compute(ebuf0, idxb0, keyb0)

            @pl.when(k < _S_PAIRS - 1)
            def _():
                _start_e(2 * k + 3, ebuf1, se1)

            _wait_g(idxb1, rows1, sg1)
            _start_g(idxb0, rows0, sg0)
            pltpu.sync_copy(rows1, acc.at[keyb1], add=True)
            return ()
        lax.fori_loop(0, _S_PAIRS, _pair, ())

        _wait_g(idxb0, rows0, sg0)
        pltpu.sync_copy(rows0, acc.at[keyb0], add=True)

        plsc.subcore_barrier()

        pltpu.sync_copy(acc.at[stripe], out_h.at[stripe, pl.ds(col, SL)])
        if p_local < PPC - 1:
            _zero_stripe()

            @pl.when(s == 0)
            def _():
                _stage(p + 1)


_scatter_scratch = [
        pltpu.VMEM((2, _EB), jnp.int32),
        pltpu.VMEM((2, _EB), jnp.int32),
        pltpu.VMEM((_EB,), jnp.int32),
        pltpu.VMEM((_EB,), jnp.int32),
        pltpu.VMEM((_EB,), jnp.int32),
        pltpu.VMEM((_EB,), jnp.int32),
        pltpu.VMEM((_EB, SL), jnp.float32),
        pltpu.VMEM((_EB, SL), jnp.float32),
        pltpu.VMEM((750, SL), jnp.float32),
        pltpu.VMEM_SHARED((NK, SL), jnp.float32),
        pltpu.VMEM_SHARED((N, SL), jnp.float32),
        pltpu.SemaphoreType.DMA,
        pltpu.SemaphoreType.DMA,
        pltpu.SemaphoreType.DMA,
        pltpu.SemaphoreType.DMA,
]

_scatter_mesh = plsc.VectorSubcoreMesh(core_axis_name="c",
                                       subcore_axis_name="s")

_scatter_counts_fn = pl.kernel(
    functools.partial(_scatter_body, True),
    out_type=(jax.ShapeDtypeStruct((NK, D), jnp.float32),
              jax.ShapeDtypeStruct((NK, LANES), jnp.float32)),
    mesh=_scatter_mesh,
    compiler_params=pltpu.CompilerParams(use_tc_tiling_on_sc=False),
    scratch_types=_scatter_scratch,
)

_scatter_fn = pl.kernel(
    functools.partial(_scatter_body, False),
    out_type=(jax.ShapeDtypeStruct((NK, D), jnp.float32),
              jax.ShapeDtypeStruct((NK, LANES), jnp.float32)),
    mesh=_scatter_mesh,
    compiler_params=pltpu.CompilerParams(use_tc_tiling_on_sc=False),
    scratch_types=_scatter_scratch,
)


_BN = 1000


def _layer_math(xb, s_ref, c_ref, root_ref, w_ref, b_ref):
    acc = jnp.dot(xb, root_ref[...], preferred_element_type=jnp.float32)
    acc = acc + b_ref[...]
    for r in range(R):
        cnt = c_ref[r][:, 0:1]
        inv = 1.0 / jnp.maximum(cnt, 1.0)
        acc = acc + jnp.dot(s_ref[r] * inv, w_ref[r],
                            preferred_element_type=jnp.float32)
    return jnp.maximum(acc, 0.0)


def _l1_body(x_ref, s_ref, c_ref, root_ref, w_ref, b_ref, o_ref):
    o_ref[...] = _layer_math(x_ref[...], s_ref, c_ref, root_ref, w_ref, b_ref)


def _l2_body(h_ref, s_ref, c_ref, root_ref, w_ref, b_ref, batch_ref, o_ref,
             sum_s, cnt_s):
    i = pl.program_id(0)
    h2 = _layer_math(h_ref[...], s_ref, c_ref, root_ref, w_ref, b_ref)

    bids = batch_ref[0]
    p_oh = (bids == lax.broadcasted_iota(jnp.int32, (G, _BN), 0))
    p_oh = p_oh.astype(jnp.float32)

    @pl.when(i == 0)
    def _():
        sum_s[...] = jnp.zeros_like(sum_s)
        cnt_s[...] = jnp.zeros_like(cnt_s)

    sum_s[...] += jnp.dot(p_oh, h2, preferred_element_type=jnp.float32)
    cnt_s[...] += jnp.sum(p_oh, axis=1, keepdims=True)

    @pl.when(i == pl.num_programs(0) - 1)
    def _():
        o_ref[...] = sum_s[...] / jnp.maximum(cnt_s[...], 1.0)


def _run_l1(x, s1, cnt, root1, w1, b1):
    return pl.pallas_call(
        _l1_body,
        grid=(N // _BN,),
        in_specs=[
            pl.BlockSpec((_BN, D), lambda i: (i, 0)),
            pl.BlockSpec((R, _BN, D), lambda i: (0, i, 0)),
            pl.BlockSpec((R, _BN, LANES), lambda i: (0, i, 0)),
            pl.BlockSpec((D, D), lambda i: (0, 0)),
            pl.BlockSpec((R, D, D), lambda i: (0, 0, 0)),
            pl.BlockSpec((1, D), lambda i: (0, 0)),
        ],
        out_specs=pl.BlockSpec((_BN, D), lambda i: (i, 0)),
        out_shape=jax.ShapeDtypeStruct((N, D), jnp.float32),
    )(x, s1, cnt, root1, w1, b1)


def _run_l2(h, s2, cnt, root2, w2, b2, batch3):
    return pl.pallas_call(
        _l2_body,
        grid=(N // _BN,),
        in_specs=[
            pl.BlockSpec((_BN, D), lambda i: (i, 0)),
            pl.BlockSpec((R, _BN, D), lambda i: (0, i, 0)),
            pl.BlockSpec((R, _BN, LANES), lambda i: (0, i, 0)),
            pl.BlockSpec((D, D), lambda i: (0, 0)),
            pl.BlockSpec((R, D, D), lambda i: (0, 0, 0)),
            pl.BlockSpec((1, D), lambda i: (0, 0)),
            pl.BlockSpec((1, 1, _BN), lambda i: (i, 0, 0)),
        ],
        out_specs=pl.BlockSpec((G, D), lambda i: (0, 0)),
        out_shape=jax.ShapeDtypeStruct((G, D), jnp.float32),
        scratch_shapes=[
            pltpu.VMEM((G, D), jnp.float32),
            pltpu.VMEM((G, 1), jnp.float32),
        ],
    )(h, s2, cnt, root2, w2, b2, batch3)


def kernel(x, edge_index, edge_type, batch, W1, root1, b1, W2, root2, b2):
    src = edge_index[0].astype(jnp.int32)
    dst = edge_index[1].astype(jnp.int32)
    et = edge_type.astype(jnp.int32)
    batch3 = batch.astype(jnp.int32).reshape(N // _BN, 1, _BN)

    edges = jnp.stack([src, et * N + dst])

    s1, cnt = _scatter_counts_fn(x, edges)
    s1 = s1.reshape(R, N, D)
    cnt = cnt.reshape(R, N, LANES)

    h = _run_l1(x, s1, cnt, root1, W1, b1.reshape(1, D))

    s2, _ = _scatter_fn(h, edges)
    s2 = s2.reshape(R, N, D)

    return _run_l2(h, s2, cnt, root2, W2, b2.reshape(1, D), batch3)
